# baseline (speedup 1.0000x reference)
import jax
import jax.numpy as jnp
import numpy as np
from jax.experimental import pallas as pl

CELL_SIZE = 0.01
MAX_NEIGHBORS = 64
TABLE_SIZE = 1000000
RADIUS = 0.01
N_CELLS = 27
Q_BLK = 128


def _hash_coords(coords):
    p1 = np.int32(73856093)
    p2 = np.int32(19349663)
    p3 = np.int32(83492791)
    h = coords[..., 0] * p1 + coords[..., 1] * p2 + coords[..., 2] * p3
    return jnp.abs(h) % TABLE_SIZE


def _query_kernel(qpos_ref, cand_ref, px_ref, py_ref, pz_ref, cnt_ref,
                  nbr_ref, nv_ref):
    qx = qpos_ref[:, 0:1]
    qy = qpos_ref[:, 1:2]
    qz = qpos_ref[:, 2:3]
    out = jnp.zeros((Q_BLK, MAX_NEIGHBORS), jnp.float32)
    base = jnp.zeros((Q_BLK, 1), jnp.float32)
    slot = jax.lax.broadcasted_iota(jnp.int32, (Q_BLK, MAX_NEIGHBORS), 1)
    jslot = jax.lax.broadcasted_iota(
        jnp.int32, (Q_BLK, MAX_NEIGHBORS, MAX_NEIGHBORS), 2)
    row = jax.lax.broadcasted_iota(jnp.int32, (MAX_NEIGHBORS, MAX_NEIGHBORS), 0)
    col = jax.lax.broadcasted_iota(jnp.int32, (MAX_NEIGHBORS, MAX_NEIGHBORS), 1)
    tri = (row <= col).astype(jnp.float32)
    def body(c, carry):
        out, base = carry
        ids = cand_ref[c]
        dx = px_ref[c] - qx
        dy = py_ref[c] - qy
        dz = pz_ref[c] - qz
        dist = jnp.sqrt((dx * dx + dy * dy) + dz * dz)
        cnt = cnt_ref[c]
        m = (slot < cnt) & (dist <= RADIUS)
        mf = m.astype(jnp.float32)
        cum = jnp.dot(mf, tri, preferred_element_type=jnp.float32)
        pos = (base + cum).astype(jnp.int32) - 1
        posm = jnp.where(m, pos, -2)
        oh = posm[:, :, None] == jslot
        idp = ids.astype(jnp.float32) + 1.0
        out = out + jnp.sum(oh.astype(jnp.float32) * idp[:, :, None], axis=1)
        base = base + cum[:, -1:]
        return out, base

    out, base = jax.lax.fori_loop(0, N_CELLS, body, (out, base))
    nv = jnp.minimum(base, float(MAX_NEIGHBORS))
    nbr_ref[:, :] = out.astype(jnp.int32) - 1
    nv_ref[:, :] = nv.astype(jnp.int32)


def kernel(positions, query_positions):
    N = positions.shape[0]
    Q = query_positions.shape[0]
    h = _hash_coords(jnp.floor(positions / CELL_SIZE).astype(jnp.int32))
    order = jnp.argsort(h)
    sorted_h = h[order]
    psx = positions[order, 0]
    psy = positions[order, 1]
    psz = positions[order, 2]

    d = jnp.arange(-1, 2)
    offs = jnp.stack(jnp.meshgrid(d, d, d, indexing='ij'),
                     axis=-1).reshape(-1, 3).astype(jnp.float32)
    cells = query_positions[:, None, :] / CELL_SIZE + offs[None, :, :]
    npos = cells * CELL_SIZE
    coords = jnp.floor(npos / CELL_SIZE).astype(jnp.int32)
    qh = _hash_coords(coords)

    start = jnp.searchsorted(sorted_h, qh)
    end = jnp.searchsorted(sorted_h, qh, side='right')
    cnt = jnp.minimum(end - start, MAX_NEIGHBORS).astype(jnp.int32)

    s = jnp.arange(MAX_NEIGHBORS)
    gidx = jnp.clip(start.T[:, :, None] + s[None, None, :], 0, N - 1)
    cand = jnp.take(order, gidx).astype(jnp.int32)
    px = jnp.take(psx, gidx)
    py = jnp.take(psy, gidx)
    pz = jnp.take(psz, gidx)
    cntT = cnt.T[:, :, None]

    grid = (Q // Q_BLK,)
    cell_spec = pl.BlockSpec((N_CELLS, Q_BLK, MAX_NEIGHBORS),
                             lambda i: (0, i, 0))
    nbr, nv = pl.pallas_call(
        _query_kernel,
        grid=grid,
        in_specs=[
            pl.BlockSpec((Q_BLK, 3), lambda i: (i, 0)),
            cell_spec,
            cell_spec,
            cell_spec,
            cell_spec,
            pl.BlockSpec((N_CELLS, Q_BLK, 1), lambda i: (0, i, 0)),
        ],
        out_specs=[
            pl.BlockSpec((Q_BLK, MAX_NEIGHBORS), lambda i: (i, 0)),
            pl.BlockSpec((Q_BLK, 1), lambda i: (i, 0)),
        ],
        out_shape=[
            jax.ShapeDtypeStruct((Q, MAX_NEIGHBORS), jnp.int32),
            jax.ShapeDtypeStruct((Q, 1), jnp.int32),
        ],
    )(query_positions, cand, px, py, pz, cntT)
    return nbr, nv.reshape(Q)


# speedup vs baseline: 1.0023x; 1.0023x over previous
import jax
import jax.numpy as jnp
import numpy as np
from jax.experimental import pallas as pl

CELL_SIZE = 0.01
MAX_NEIGHBORS = 64
TABLE_SIZE = 1000000
RADIUS = 0.01
N_CELLS = 27
Q_BLK = 128


def _hash_coords(coords):
    p1 = np.int32(73856093)
    p2 = np.int32(19349663)
    p3 = np.int32(83492791)
    h = coords[..., 0] * p1 + coords[..., 1] * p2 + coords[..., 2] * p3
    return jnp.abs(h) % TABLE_SIZE


def _query_kernel(qpos_ref, cand_ref, px_ref, py_ref, pz_ref, cnt_ref,
                  nbr_ref, nv_ref):
    qx = qpos_ref[:, 0:1]
    qy = qpos_ref[:, 1:2]
    qz = qpos_ref[:, 2:3]
    out = jnp.zeros((Q_BLK, MAX_NEIGHBORS), jnp.float32)
    base = jnp.zeros((Q_BLK, 1), jnp.float32)
    slot = jax.lax.broadcasted_iota(jnp.int32, (Q_BLK, MAX_NEIGHBORS), 1)
    jslot = jax.lax.broadcasted_iota(
        jnp.int32, (Q_BLK, MAX_NEIGHBORS, MAX_NEIGHBORS), 1)
    row = jax.lax.broadcasted_iota(jnp.int32, (MAX_NEIGHBORS, MAX_NEIGHBORS), 0)
    col = jax.lax.broadcasted_iota(jnp.int32, (MAX_NEIGHBORS, MAX_NEIGHBORS), 1)
    tri = (row <= col).astype(jnp.float32)
    def body(c, carry):
        out, base = carry
        ids = cand_ref[c]
        dx = px_ref[c] - qx
        dy = py_ref[c] - qy
        dz = pz_ref[c] - qz
        dist = jnp.sqrt((dx * dx + dy * dy) + dz * dz)
        cnt = cnt_ref[c]
        m = (slot < cnt) & (dist <= RADIUS)
        mf = m.astype(jnp.float32)
        cum = jnp.dot(mf, tri, preferred_element_type=jnp.float32)
        pos = (base + cum).astype(jnp.int32) - 1
        posm = jnp.where(m, pos, -2)
        oh = posm[:, None, :] == jslot
        idp = ids.astype(jnp.float32) + 1.0
        out = out + jnp.sum(oh.astype(jnp.float32) * idp[:, None, :], axis=2)
        base = base + cum[:, -1:]
        return out, base

    out, base = jax.lax.fori_loop(0, N_CELLS, body, (out, base))
    nv = jnp.minimum(base, float(MAX_NEIGHBORS))
    nbr_ref[:, :] = out.astype(jnp.int32) - 1
    nv_ref[:, :] = nv.astype(jnp.int32)


def kernel(positions, query_positions):
    N = positions.shape[0]
    Q = query_positions.shape[0]
    h = _hash_coords(jnp.floor(positions / CELL_SIZE).astype(jnp.int32))
    order = jnp.argsort(h)
    sorted_h = h[order]
    psx = positions[order, 0]
    psy = positions[order, 1]
    psz = positions[order, 2]

    d = jnp.arange(-1, 2)
    offs = jnp.stack(jnp.meshgrid(d, d, d, indexing='ij'),
                     axis=-1).reshape(-1, 3).astype(jnp.float32)
    cells = query_positions[:, None, :] / CELL_SIZE + offs[None, :, :]
    npos = cells * CELL_SIZE
    coords = jnp.floor(npos / CELL_SIZE).astype(jnp.int32)
    qh = _hash_coords(coords)

    start = jnp.searchsorted(sorted_h, qh)
    end = jnp.searchsorted(sorted_h, qh, side='right')
    cnt = jnp.minimum(end - start, MAX_NEIGHBORS).astype(jnp.int32)

    s = jnp.arange(MAX_NEIGHBORS)
    gidx = jnp.clip(start.T[:, :, None] + s[None, None, :], 0, N - 1)
    cand = jnp.take(order, gidx).astype(jnp.int32)
    px = jnp.take(psx, gidx)
    py = jnp.take(psy, gidx)
    pz = jnp.take(psz, gidx)
    cntT = cnt.T[:, :, None]

    grid = (Q // Q_BLK,)
    cell_spec = pl.BlockSpec((N_CELLS, Q_BLK, MAX_NEIGHBORS),
                             lambda i: (0, i, 0))
    nbr, nv = pl.pallas_call(
        _query_kernel,
        grid=grid,
        in_specs=[
            pl.BlockSpec((Q_BLK, 3), lambda i: (i, 0)),
            cell_spec,
            cell_spec,
            cell_spec,
            cell_spec,
            pl.BlockSpec((N_CELLS, Q_BLK, 1), lambda i: (0, i, 0)),
        ],
        out_specs=[
            pl.BlockSpec((Q_BLK, MAX_NEIGHBORS), lambda i: (i, 0)),
            pl.BlockSpec((Q_BLK, 1), lambda i: (i, 0)),
        ],
        out_shape=[
            jax.ShapeDtypeStruct((Q, MAX_NEIGHBORS), jnp.int32),
            jax.ShapeDtypeStruct((Q, 1), jnp.int32),
        ],
    )(query_positions, cand, px, py, pz, cntT)
    return nbr, nv.reshape(Q)
